# baseline (speedup 1.0000x reference)
import functools

import jax
import jax.numpy as jnp
from jax import lax
from jax.experimental import pallas as pl
from jax.experimental.pallas import tpu as pltpu
from jax.experimental.pallas import tpu_sc as plsc

_KERNEL_NUM = 64
_SIZE = 1048576
_LANES = 16


def _make_sc_kernel():
    info = plsc.get_sparse_core_info()
    num_workers = info.num_cores * info.num_subcores
    chunk = _SIZE // num_workers

    mesh = plsc.VectorSubcoreMesh(core_axis_name="c", subcore_axis_name="s")

    @functools.partial(
        pl.kernel,
        out_type=jax.ShapeDtypeStruct((1, _SIZE), jnp.float32),
        mesh=mesh,
        scratch_types=[
            pltpu.VMEM((_LANES,), jnp.float32),
            pltpu.VMEM((chunk,), jnp.float32),
            pltpu.VMEM((chunk,), jnp.float32),
            pltpu.SemaphoreType.DMA,
            pltpu.SemaphoreType.DMA,
        ],
    )
    def blend(lam_hbm, const_hbm, pivots_hbm, out_hbm, lam_v, lbuf, rbuf,
              lsem, rsem):
        del pivots_hbm
        wid = lax.axis_index("s") * info.num_cores + lax.axis_index("c")
        base = wid * chunk

        pltpu.sync_copy(lam_hbm, lam_v.at[pl.ds(0, 1)])
        lam_s = lam_v[...][0] * jnp.float32(0.99999)

        scaled = lam_s * jnp.float32(_KERNEL_NUM - 1)
        left = scaled.astype(jnp.int32)
        left = jnp.minimum(jnp.maximum(left, 0), _KERNEL_NUM - 2)
        dist = (left.astype(jnp.float32) + jnp.float32(1.0)) - scaled

        lcp = pltpu.async_copy(
            const_hbm.at[left, pl.ds(base, chunk)], lbuf, lsem)
        rcp = pltpu.async_copy(
            const_hbm.at[left + 1, pl.ds(base, chunk)], rbuf, rsem)
        lcp.wait()
        rcp.wait()

        one_minus = jnp.float32(1.0) - dist

        def body(i, _):
            sl = pl.ds(i * _LANES, _LANES)
            lbuf[sl] = lbuf[sl] * dist + rbuf[sl] * one_minus
            return 0

        lax.fori_loop(0, chunk // _LANES, body, 0)

        pltpu.sync_copy(lbuf, out_hbm.at[0, pl.ds(base, chunk)])

    return blend


_blend = _make_sc_kernel()


def kernel(lam, const, pivots):
    return _blend(lam, const, pivots)


# speedup vs baseline: 1.3434x; 1.3434x over previous
import functools

import jax
import jax.numpy as jnp
from jax import lax
from jax.experimental import pallas as pl
from jax.experimental.pallas import tpu as pltpu
from jax.experimental.pallas import tpu_sc as plsc

_KERNEL_NUM = 64
_SIZE = 1048576
_LANES = 16


def _make_sc_kernel():
    info = plsc.get_sparse_core_info()
    num_workers = info.num_cores * info.num_subcores
    chunk = _SIZE // num_workers

    mesh = plsc.VectorSubcoreMesh(core_axis_name="c", subcore_axis_name="s")

    @functools.partial(
        pl.kernel,
        out_type=jax.ShapeDtypeStruct((1, _SIZE), jnp.float32),
        mesh=mesh,
        scratch_types=[
            pltpu.VMEM((_LANES,), jnp.float32),
            pltpu.VMEM((chunk,), jnp.float32),
            pltpu.VMEM((chunk,), jnp.float32),
            pltpu.VMEM((chunk,), jnp.float32),
            pltpu.SemaphoreType.DMA,
            pltpu.SemaphoreType.DMA,
        ],
    )
    def blend(lam_hbm, const_hbm, pivots_hbm, out_hbm, lam_v, lbuf, rbuf,
              obuf, lsem, rsem):
        del pivots_hbm
        wid = lax.axis_index("s") * info.num_cores + lax.axis_index("c")
        base = wid * chunk

        pltpu.sync_copy(lam_hbm, lam_v.at[pl.ds(0, 1)])
        lam_s = lam_v[...][0] * jnp.float32(0.99999)

        scaled = lam_s * jnp.float32(_KERNEL_NUM - 1)
        left = scaled.astype(jnp.int32)
        left = jnp.minimum(jnp.maximum(left, 0), _KERNEL_NUM - 2)
        dist = (left.astype(jnp.float32) + jnp.float32(1.0)) - scaled

        lcp = pltpu.async_copy(
            const_hbm.at[left, pl.ds(base, chunk)], lbuf, lsem)
        rcp = pltpu.async_copy(
            const_hbm.at[left + 1, pl.ds(base, chunk)], rbuf, rsem)
        lcp.wait()
        rcp.wait()

        one_minus = jnp.float32(1.0) - dist

        @plsc.parallel_loop(0, chunk, step=_LANES, unroll=8)
        def _(i):
            sl = pl.ds(i, _LANES)
            obuf[sl] = lbuf[sl] * dist + rbuf[sl] * one_minus

        pltpu.sync_copy(obuf, out_hbm.at[0, pl.ds(base, chunk)])

    return blend


_blend = _make_sc_kernel()


def kernel(lam, const, pivots):
    return _blend(lam, const, pivots)
